# baseline (device time: 30354 ns/iter reference)
import jax
import jax.numpy as jnp
from jax import lax
from jax.experimental import pallas as pl
from jax.experimental.pallas import tpu as pltpu

N_DEV = 4
N_LAYERS = 3
CHUNKS = 4


def kernel(x, Win0, Wout0, Win1, Wout1, Win2, Wout2):
    b, _ = x.shape
    h_dim = Win0.shape[1]
    out_cols = Wout0.shape[1]
    cw = h_dim // CHUNKS

    def body(x_ref, win0, wout0, win1, wout1, win2, wout2,
             out_ref, comm_ref, send_sems, recv_sems):
        my = lax.axis_index("i")

        barrier_sem = pltpu.get_barrier_semaphore()
        for d in range(1, N_DEV):
            pl.semaphore_signal(
                barrier_sem, inc=1,
                device_id=((my + d) % N_DEV,),
                device_id_type=pl.DeviceIdType.MESH,
            )
        pl.semaphore_wait(barrier_sem, N_DEV - 1)

        wins = (win0, win1, win2)
        wouts = (wout0, wout1, wout2)

        def make_rdma(l, d, c):
            return pltpu.make_async_remote_copy(
                src_ref=comm_ref.at[l, 0, c],
                dst_ref=comm_ref.at[l, d, c],
                send_sem=send_sems.at[l, d - 1, c],
                recv_sem=recv_sems.at[l, d - 1, c],
                device_id=((my + d) % N_DEV,),
                device_id_type=pl.DeviceIdType.MESH,
            )

        x_cur = x_ref[:, :]
        for l in range(N_LAYERS):
            rdmas = []
            x_b = x_cur.astype(jnp.bfloat16)
            for c in range(CHUNKS):
                partial_c = jnp.dot(
                    x_b, wins[l][:, c * cw:(c + 1) * cw].astype(jnp.bfloat16),
                    preferred_element_type=jnp.float32,
                )
                comm_ref[l, 0, c, :, :] = partial_c.astype(jnp.bfloat16)
                for d in range(1, N_DEV):
                    rdma = make_rdma(l, d, c)
                    rdma.start()
                    rdmas.append(rdma)
            x_next = None
            for c in range(CHUNKS):
                acc = comm_ref[l, 0, c, :, :].astype(jnp.float32)
                for d in range(1, N_DEV):
                    rdmas[c * (N_DEV - 1) + (d - 1)].wait_recv()
                    acc = acc + comm_ref[l, d, c, :, :].astype(jnp.float32)
                h_c = jnp.maximum(acc, 0.0).astype(jnp.bfloat16)
                part = jnp.dot(
                    h_c, wouts[l][c * cw:(c + 1) * cw, :].astype(jnp.bfloat16),
                    preferred_element_type=jnp.float32,
                )
                x_next = part if x_next is None else x_next + part
            x_cur = x_next
            for r in rdmas:
                r.wait_send()
        out_ref[:, :] = x_cur

    return pl.pallas_call(
        body,
        out_shape=jax.ShapeDtypeStruct((b, out_cols), jnp.float32),
        in_specs=[pl.BlockSpec(memory_space=pltpu.VMEM)] * 7,
        out_specs=pl.BlockSpec(memory_space=pltpu.VMEM),
        scratch_shapes=[
            pltpu.VMEM((N_LAYERS, N_DEV, CHUNKS, b, cw), jnp.bfloat16),
            pltpu.SemaphoreType.DMA((N_LAYERS, N_DEV - 1, CHUNKS)),
            pltpu.SemaphoreType.DMA((N_LAYERS, N_DEV - 1, CHUNKS)),
        ],
        compiler_params=pltpu.CompilerParams(collective_id=0),
    )(x, Win0, Wout0, Win1, Wout1, Win2, Wout2)


# device time: 30073 ns/iter; 1.0093x vs baseline; 1.0093x over previous
import jax
import jax.numpy as jnp
from jax import lax
from jax.experimental import pallas as pl
from jax.experimental.pallas import tpu as pltpu

N_DEV = 4
N_LAYERS = 3
CHUNKS = 4


def kernel(x, Win0, Wout0, Win1, Wout1, Win2, Wout2):
    b, _ = x.shape
    h_dim = Win0.shape[1]
    out_cols = Wout0.shape[1]
    cw = h_dim // CHUNKS

    def body(x_ref, win0, wout0, win1, wout1, win2, wout2,
             out_ref, comm_ref, send_sems, recv_sems):
        my = lax.axis_index("i")

        barrier_sem = pltpu.get_barrier_semaphore()
        for d in range(1, N_DEV):
            pl.semaphore_signal(
                barrier_sem, inc=1,
                device_id=((my + d) % N_DEV,),
                device_id_type=pl.DeviceIdType.MESH,
            )
        pl.semaphore_wait(barrier_sem, N_DEV - 1)

        wins = (win0, win1, win2)
        wouts = (wout0, wout1, wout2)

        def make_rdma(l, d, c):
            return pltpu.make_async_remote_copy(
                src_ref=comm_ref.at[l, 0, c],
                dst_ref=comm_ref.at[l, d, c],
                send_sem=send_sems.at[l, d - 1, c],
                recv_sem=recv_sems.at[l, d - 1, c],
                device_id=((my + d) % N_DEV,),
                device_id_type=pl.DeviceIdType.MESH,
            )

        x_cur = x_ref[:, :]
        for l in range(N_LAYERS):
            rdmas = []
            x_b = x_cur.astype(jnp.bfloat16)
            for c in range(CHUNKS):
                partial_c = jnp.dot(
                    x_b, wins[l][:, c * cw:(c + 1) * cw].astype(jnp.bfloat16),
                    preferred_element_type=jnp.float32,
                )
                comm_ref[l, 0, c, :, :] = partial_c.astype(jnp.bfloat16)
                chunk_rdmas = {}
                for d in (2, 1, 3):
                    rdma = make_rdma(l, d, c)
                    rdma.start()
                    chunk_rdmas[d] = rdma
                for d in range(1, N_DEV):
                    rdmas.append(chunk_rdmas[d])
            x_next = None
            for c in range(CHUNKS):
                acc = comm_ref[l, 0, c, :, :].astype(jnp.float32)
                for d in range(1, N_DEV):
                    rdmas[c * (N_DEV - 1) + (d - 1)].wait_recv()
                    acc = acc + comm_ref[l, d, c, :, :].astype(jnp.float32)
                h_c = jnp.maximum(acc, 0.0).astype(jnp.bfloat16)
                part = jnp.dot(
                    h_c, wouts[l][c * cw:(c + 1) * cw, :].astype(jnp.bfloat16),
                    preferred_element_type=jnp.float32,
                )
                x_next = part if x_next is None else x_next + part
            x_cur = x_next
            for r in rdmas:
                r.wait_send()
        out_ref[:, :] = x_cur

    return pl.pallas_call(
        body,
        out_shape=jax.ShapeDtypeStruct((b, out_cols), jnp.float32),
        in_specs=[pl.BlockSpec(memory_space=pltpu.VMEM)] * 7,
        out_specs=pl.BlockSpec(memory_space=pltpu.VMEM),
        scratch_shapes=[
            pltpu.VMEM((N_LAYERS, N_DEV, CHUNKS, b, cw), jnp.bfloat16),
            pltpu.SemaphoreType.DMA((N_LAYERS, N_DEV - 1, CHUNKS)),
            pltpu.SemaphoreType.DMA((N_LAYERS, N_DEV - 1, CHUNKS)),
        ],
        compiler_params=pltpu.CompilerParams(collective_id=0),
    )(x, Win0, Wout0, Win1, Wout1, Win2, Wout2)


# device time: 27135 ns/iter; 1.1186x vs baseline; 1.1083x over previous
import jax
import jax.numpy as jnp
from jax import lax
from jax.experimental import pallas as pl
from jax.experimental.pallas import tpu as pltpu

N_DEV = 4
N_LAYERS = 3
N_HALF = 2


def kernel(x, Win0, Wout0, Win1, Wout1, Win2, Wout2):
    b, _ = x.shape
    h_dim = Win0.shape[1]
    out_cols = Wout0.shape[1]
    bh = b // N_HALF

    def body(x_ref, win0, wout0, win1, wout1, win2, wout2,
             out_ref, comm_ref, send_sems, recv_sems):
        my = lax.axis_index("i")

        barrier_sem = pltpu.get_barrier_semaphore()
        for d in range(1, N_DEV):
            pl.semaphore_signal(
                barrier_sem, inc=1,
                device_id=((my + d) % N_DEV,),
                device_id_type=pl.DeviceIdType.MESH,
            )
        pl.semaphore_wait(barrier_sem, N_DEV - 1)

        wins = (win0, win1, win2)
        wouts = (wout0, wout1, wout2)
        rdmas = {}

        def compute_send(l, h, x_half):
            p = jnp.dot(
                x_half.astype(jnp.bfloat16),
                wins[l][:, :].astype(jnp.bfloat16),
                preferred_element_type=jnp.float32,
            )
            comm_ref[l, h, 0, :, :] = p.astype(jnp.bfloat16)
            for d in (2, 1, 3):
                r = pltpu.make_async_remote_copy(
                    src_ref=comm_ref.at[l, h, 0],
                    dst_ref=comm_ref.at[l, h, d],
                    send_sem=send_sems.at[l, h, d - 1],
                    recv_sem=recv_sems.at[l, h, d - 1],
                    device_id=((my + d) % N_DEV,),
                    device_id_type=pl.DeviceIdType.MESH,
                )
                r.start()
                rdmas[(l, h, d)] = r

        def recv_mm2(l, h):
            acc = comm_ref[l, h, 0, :, :].astype(jnp.float32)
            for d in (1, 3, 2):
                rdmas[(l, h, d)].wait_recv()
                acc = acc + comm_ref[l, h, d, :, :].astype(jnp.float32)
            hidden = jnp.maximum(acc, 0.0).astype(jnp.bfloat16)
            return jnp.dot(
                hidden, wouts[l][:, :].astype(jnp.bfloat16),
                preferred_element_type=jnp.float32,
            )

        xa = x_ref[:bh, :]
        xb = x_ref[bh:, :]
        compute_send(0, 0, xa)
        compute_send(0, 1, xb)
        for l in range(N_LAYERS - 1):
            xa = recv_mm2(l, 0)
            compute_send(l + 1, 0, xa)
            xb = recv_mm2(l, 1)
            compute_send(l + 1, 1, xb)
        out_ref[:bh, :] = recv_mm2(N_LAYERS - 1, 0)
        out_ref[bh:, :] = recv_mm2(N_LAYERS - 1, 1)

        for r in rdmas.values():
            r.wait_send()

    return pl.pallas_call(
        body,
        out_shape=jax.ShapeDtypeStruct((b, out_cols), jnp.float32),
        in_specs=[pl.BlockSpec(memory_space=pltpu.VMEM)] * 7,
        out_specs=pl.BlockSpec(memory_space=pltpu.VMEM),
        scratch_shapes=[
            pltpu.VMEM((N_LAYERS, N_HALF, N_DEV, bh, h_dim), jnp.bfloat16),
            pltpu.SemaphoreType.DMA((N_LAYERS, N_HALF, N_DEV - 1)),
            pltpu.SemaphoreType.DMA((N_LAYERS, N_HALF, N_DEV - 1)),
        ],
        compiler_params=pltpu.CompilerParams(collective_id=0),
    )(x, Win0, Wout0, Win1, Wout1, Win2, Wout2)
